# baseline (speedup 1.0000x reference)
import jax
import jax.numpy as jnp
from jax import lax
from jax.experimental import pallas as pl
from jax.experimental.pallas import tpu as pltpu

_B = 256
_HW = 28
_BT = 8


def _net_body(dense_ref, mask_ref, mask2_ref, w1_ref, w2_ref, w3_ref,
              s1_ref, t1_ref, s2_ref, t2_ref, s3_ref, t3_ref, fc1w_ref,
              fc1b_ref, fc2w_ref, fc2b_ref, out_ref, dp_ref, h1p_ref):
    bt = dense_ref.shape[0]

    @pl.when(pl.program_id(0) == 0)
    def _zero():
        dp_ref[...] = jnp.zeros_like(dp_ref)
        h1p_ref[...] = jnp.zeros_like(h1p_ref)

    m = mask_ref[...]
    dp_ref[:, 1:_HW + 1, 1:_HW + 1] = dense_ref[...]
    dp = dp_ref[...]

    h1 = jnp.zeros((bt, _HW, _HW, 32), jnp.float32)
    for ky in range(3):
        for kx in range(3):
            tap = dp[:, ky:ky + _HW, kx:kx + _HW]
            h1 = h1 + tap[..., None] * w1_ref[3 * ky + kx]
    h1 = jnp.maximum(h1 * s1_ref[0] + t1_ref[0], 0.0) * m[..., None]
    h1p_ref[:, 1:_HW + 1, 1:_HW + 1, :] = h1

    acc = jnp.zeros((bt * _HW * _HW, 64), jnp.float32)
    for ky in range(3):
        for kx in range(3):
            tap = h1p_ref[:, ky:ky + _HW, kx:kx + _HW, :].reshape(
                bt * _HW * _HW, 32)
            acc = acc + jnp.dot(tap, w2_ref[3 * ky + kx],
                                preferred_element_type=jnp.float32)
    h2 = (jnp.maximum(acc * s2_ref[0] + t2_ref[0], 0.0)
          .reshape(bt, _HW, _HW, 64) * m[..., None])

    h2v = h2.reshape(bt, 14, 2, 14, 2, 64)
    acc3 = jnp.zeros((bt * 14 * 14, 64), jnp.float32)
    for dy in range(2):
        for dx in range(2):
            tap = h2v[:, :, dy, :, dx, :].reshape(bt * 14 * 14, 64)
            acc3 = acc3 + jnp.dot(tap, w3_ref[2 * dy + dx],
                                  preferred_element_type=jnp.float32)
    m2 = mask2_ref[...]
    h3 = (jnp.maximum(acc3 * s3_ref[0] + t3_ref[0], 0.0)
          .reshape(bt, 14, 14, 64) * m2[..., None])

    h3t = jnp.transpose(h3.reshape(bt, 196, 64), (0, 2, 1)).reshape(
        bt, 64 * 196)
    z = jnp.maximum(jnp.dot(h3t, fc1w_ref[...],
                            preferred_element_type=jnp.float32)
                    + fc1b_ref[0], 0.0)
    z2 = jnp.dot(z, fc2w_ref[...],
                 preferred_element_type=jnp.float32) + fc2b_ref[0]
    mx = jnp.max(z2, axis=1, keepdims=True)
    lse = jnp.log(jnp.sum(jnp.exp(z2 - mx), axis=1, keepdims=True)) + mx
    out_ref[...] = z2 - lse


def _run_net(dense, mask, mask2, w1r, w2r, w3r, s1, t1, s2, t2, s3, t3,
             fc1_w, fc1_b, fc2_w, fc2_b):
    nsteps = _B // _BT
    full = lambda shape: pl.BlockSpec(shape, lambda i: (0,) * len(shape))
    return pl.pallas_call(
        _net_body,
        grid=(nsteps,),
        in_specs=[
            pl.BlockSpec((_BT, _HW, _HW), lambda i: (i, 0, 0)),
            pl.BlockSpec((_BT, _HW, _HW), lambda i: (i, 0, 0)),
            pl.BlockSpec((_BT, 14, 14), lambda i: (i, 0, 0)),
            full((9, 32)),
            full((9, 32, 64)),
            full((4, 64, 64)),
            full((1, 32)), full((1, 32)),
            full((1, 64)), full((1, 64)),
            full((1, 64)), full((1, 64)),
            full((64 * 196, 128)),
            full((1, 128)),
            full((128, 10)),
            full((1, 10)),
        ],
        out_specs=pl.BlockSpec((_BT, 10), lambda i: (i, 0)),
        out_shape=jax.ShapeDtypeStruct((_B, 10), jnp.float32),
        scratch_shapes=[
            pltpu.VMEM((_BT, _HW + 2, _HW + 2), jnp.float32),
            pltpu.VMEM((_BT, _HW + 2, _HW + 2, 32), jnp.float32),
        ],
    )(dense, mask, mask2, w1r, w2r, w3r, s1, t1, s2, t2, s3, t3,
      fc1_w, fc1_b, fc2_w, fc2_b)


def kernel(features, indices, w1, g1, b1, m1, v1, w2, g2, b2, m2, v2,
           w3, g3, b3, m3, v3, fc1_w, fc1_b, fc2_w, fc2_b):
    bi, yi, xi = indices[:, 0], indices[:, 1], indices[:, 2]
    dense = jnp.zeros((_B, _HW, _HW), jnp.float32).at[bi, yi, xi].set(
        features[:, 0])
    mask = jnp.zeros((_B, _HW, _HW), jnp.float32).at[bi, yi, xi].set(1.0)
    mask2 = jnp.zeros((_B, 14, 14), jnp.float32).at[bi, yi // 2,
                                                    xi // 2].set(1.0)

    def fold(g, b, mm, vv):
        s = g * lax.rsqrt(vv + 1e-5)
        return s, b - mm * s

    s1, t1 = fold(g1, b1, m1, v1)
    s2, t2 = fold(g2, b2, m2, v2)
    s3, t3 = fold(g3, b3, m3, v3)
    return _run_net(
        dense, mask, mask2,
        w1.reshape(9, 32), w2.reshape(9, 32, 64), w3.reshape(4, 64, 64),
        s1.reshape(1, 32), t1.reshape(1, 32),
        s2.reshape(1, 64), t2.reshape(1, 64),
        s3.reshape(1, 64), t3.reshape(1, 64),
        fc1_w, fc1_b.reshape(1, 128), fc2_w, fc2_b.reshape(1, 10))


# speedup vs baseline: 1.5412x; 1.5412x over previous
import dataclasses

import jax
import jax.numpy as jnp
from jax import lax
from jax.experimental import pallas as pl
from jax.experimental.pallas import tpu as pltpu
from jax.experimental.pallas import tpu_sc as plsc

_B = 256
_HW = 28
_BT = 8

_NPTS = 38400
_CHUNK = 1536
_TILES = 32
_NCELL = _B * _HW * _HW
_NCELL2 = _B * 14 * 14
_DSL = _NCELL // _TILES
_D2SL = _NCELL2 // _TILES


def _densify_body(bi_hbm, yi_hbm, xi_hbm, f_hbm, dense_hbm, mask_hbm,
                  mask2_hbm, b_buf, y_buf, x_buf, f_buf, dense_loc,
                  mask_loc, m2_loc):
    t = lax.axis_index("c") * 16 + lax.axis_index("s")
    base_d = t * _DSL
    base_d2 = t * _D2SL
    zf = jnp.zeros((16,), jnp.float32)
    ones = jnp.full((16,), 1.0, jnp.float32)

    @pl.loop(0, _DSL // 16)
    def _(i):
        dense_loc[pl.ds(i * 16, 16)] = zf
        mask_loc[pl.ds(i * 16, 16)] = zf

    @pl.loop(0, _D2SL // 16)
    def _(i):
        m2_loc[pl.ds(i * 16, 16)] = zf

    @pl.loop(0, _NPTS // _CHUNK)
    def _(c):
        off = c * _CHUNK
        pltpu.sync_copy(bi_hbm.at[pl.ds(off, _CHUNK)], b_buf)
        pltpu.sync_copy(yi_hbm.at[pl.ds(off, _CHUNK)], y_buf)
        pltpu.sync_copy(xi_hbm.at[pl.ds(off, _CHUNK)], x_buf)
        pltpu.sync_copy(f_hbm.at[pl.ds(off, _CHUNK)], f_buf)

        @pl.loop(0, _CHUNK // 16)
        def _(v):
            sl = pl.ds(v * 16, 16)
            bv = b_buf[sl]
            yv = y_buf[sl]
            xv = x_buf[sl]
            fv = f_buf[sl]
            cell = (bv * _HW + yv) * _HW + xv - base_d
            inb = (cell >= 0) & (cell < _DSL)
            plsc.store_scatter(dense_loc, [cell], fv, mask=inb)
            plsc.store_scatter(mask_loc, [cell], ones, mask=inb)
            cell2 = (bv * 14 + (yv >> 1)) * 14 + (xv >> 1) - base_d2
            plsc.store_scatter(m2_loc, [cell2], ones, mask=inb)

    pltpu.sync_copy(dense_loc, dense_hbm.at[pl.ds(base_d, _DSL)])
    pltpu.sync_copy(mask_loc, mask_hbm.at[pl.ds(base_d, _DSL)])
    pltpu.sync_copy(m2_loc, mask2_hbm.at[pl.ds(base_d2, _D2SL)])


def _densify(bi, yi, xi, feats):
    f32 = jnp.float32
    cp = pltpu.CompilerParams()
    if "needs_layout_passes" in pltpu.CompilerParams.__dataclass_fields__:
        cp = dataclasses.replace(cp, needs_layout_passes=False)
    kern = pl.kernel(
        _densify_body,
        out_type=[jax.ShapeDtypeStruct((_NCELL,), f32),
                  jax.ShapeDtypeStruct((_NCELL,), f32),
                  jax.ShapeDtypeStruct((_NCELL2,), f32)],
        mesh=plsc.VectorSubcoreMesh(core_axis_name="c", subcore_axis_name="s"),
        scratch_types=[pltpu.VMEM((_CHUNK,), jnp.int32),
                       pltpu.VMEM((_CHUNK,), jnp.int32),
                       pltpu.VMEM((_CHUNK,), jnp.int32),
                       pltpu.VMEM((_CHUNK,), f32),
                       pltpu.VMEM((_DSL,), f32),
                       pltpu.VMEM((_DSL,), f32),
                       pltpu.VMEM((_D2SL,), f32)],
        compiler_params=cp,
    )
    return kern(bi, yi, xi, feats)


_NSTEPS = _B // _BT


def _net_body(dense_ref, mask_ref, mask2_ref, w1_ref, w2c_ref, w3_ref,
              s1_ref, t1_ref, s2_ref, t2_ref, s3_ref, t3_ref, fc1w_ref,
              fc1b_ref, fc2w_ref, fc2b_ref, out_ref, dp_ref, h1p_ref,
              h3t_ref):
    bt = dense_ref.shape[0]
    i = pl.program_id(0)

    @pl.when(i == 0)
    def _zero():
        dp_ref[...] = jnp.zeros_like(dp_ref)
        h1p_ref[...] = jnp.zeros_like(h1p_ref)

    @pl.when(i < _NSTEPS)
    def _conv_phase():
        m = mask_ref[...]
        dp_ref[:, 1:_HW + 1, 1:_HW + 1] = dense_ref[...]
        dp = dp_ref[...]

        h1 = jnp.zeros((bt, _HW, _HW, 32), jnp.float32)
        for ky in range(3):
            for kx in range(3):
                tap = dp[:, ky:ky + _HW, kx:kx + _HW]
                h1 = h1 + tap[..., None] * w1_ref[3 * ky + kx]
        h1 = jnp.maximum(h1 * s1_ref[0] + t1_ref[0], 0.0) * m[..., None]
        h1p_ref[:, 1:_HW + 1, 1:_HW + 1, :] = h1

        taps = [h1p_ref[:, ky:ky + _HW, kx:kx + _HW, :].reshape(
                    bt * _HW * _HW, 32)
                for ky in range(3) for kx in range(3)]
        cat = jnp.concatenate(taps, axis=1)
        acc = jnp.dot(cat, w2c_ref[...], preferred_element_type=jnp.float32)
        h2 = (jnp.maximum(acc * s2_ref[0] + t2_ref[0], 0.0)
              .reshape(bt, _HW, _HW, 64) * m[..., None])

        h2v = h2.reshape(bt, 14, 2, 14, 2, 64)
        acc3 = jnp.zeros((bt * 14 * 14, 64), jnp.float32)
        for dy in range(2):
            for dx in range(2):
                tap = h2v[:, :, dy, :, dx, :].reshape(bt * 14 * 14, 64)
                acc3 = acc3 + jnp.dot(tap, w3_ref[2 * dy + dx],
                                      preferred_element_type=jnp.float32)
        m2 = mask2_ref[...]
        h3 = (jnp.maximum(acc3 * s3_ref[0] + t3_ref[0], 0.0)
              .reshape(bt, 14, 14, 64) * m2[..., None])

        h3t = jnp.transpose(h3.reshape(bt, 196, 64), (0, 2, 1)).reshape(
            bt, 64 * 196)
        h3t_ref[pl.ds(i * bt, bt), :] = h3t

    @pl.when(i == _NSTEPS)
    def _fc_phase():
        z = jnp.maximum(jnp.dot(h3t_ref[...], fc1w_ref[...],
                                preferred_element_type=jnp.float32)
                        + fc1b_ref[0], 0.0)
        z2 = jnp.dot(z, fc2w_ref[...],
                     preferred_element_type=jnp.float32) + fc2b_ref[0]
        mx = jnp.max(z2, axis=1, keepdims=True)
        lse = jnp.log(jnp.sum(jnp.exp(z2 - mx), axis=1, keepdims=True)) + mx
        out_ref[...] = z2 - lse


def _run_net(dense, mask, mask2, w1r, w2c, w3r, s1, t1, s2, t2, s3, t3,
             fc1_w, fc1_b, fc2_w, fc2_b):
    full = lambda shape: pl.BlockSpec(shape, lambda i: (0,) * len(shape))
    last = _NSTEPS - 1
    tile3 = lambda i: (jnp.minimum(i, last), 0, 0)
    return pl.pallas_call(
        _net_body,
        grid=(_NSTEPS + 1,),
        in_specs=[
            pl.BlockSpec((_BT, _HW, _HW), tile3),
            pl.BlockSpec((_BT, _HW, _HW), tile3),
            pl.BlockSpec((_BT, 14, 14), tile3),
            full((9, 32)),
            full((288, 64)),
            full((4, 64, 64)),
            full((1, 32)), full((1, 32)),
            full((1, 64)), full((1, 64)),
            full((1, 64)), full((1, 64)),
            full((64 * 196, 128)),
            full((1, 128)),
            full((128, 10)),
            full((1, 10)),
        ],
        out_specs=pl.BlockSpec((_B, 10), lambda i: (0, 0)),
        out_shape=jax.ShapeDtypeStruct((_B, 10), jnp.float32),
        scratch_shapes=[
            pltpu.VMEM((_BT, _HW + 2, _HW + 2), jnp.float32),
            pltpu.VMEM((_BT, _HW + 2, _HW + 2, 32), jnp.float32),
            pltpu.VMEM((_B, 64 * 196), jnp.float32),
        ],
    )(dense, mask, mask2, w1r, w2c, w3r, s1, t1, s2, t2, s3, t3,
      fc1_w, fc1_b, fc2_w, fc2_b)


def kernel(features, indices, w1, g1, b1, m1, v1, w2, g2, b2, m2, v2,
           w3, g3, b3, m3, v3, fc1_w, fc1_b, fc2_w, fc2_b):
    dense_f, mask_f, mask2_f = _densify(indices[:, 0], indices[:, 1],
                                        indices[:, 2], features[:, 0])
    dense = dense_f.reshape(_B, _HW, _HW)
    mask = mask_f.reshape(_B, _HW, _HW)
    mask2 = mask2_f.reshape(_B, 14, 14)

    def fold(g, b, mm, vv):
        s = g * lax.rsqrt(vv + 1e-5)
        return s, b - mm * s

    s1, t1 = fold(g1, b1, m1, v1)
    s2, t2 = fold(g2, b2, m2, v2)
    s3, t3 = fold(g3, b3, m3, v3)
    return _run_net(
        dense, mask, mask2,
        w1.reshape(9, 32), w2.reshape(288, 64), w3.reshape(4, 64, 64),
        s1.reshape(1, 32), t1.reshape(1, 32),
        s2.reshape(1, 64), t2.reshape(1, 64),
        s3.reshape(1, 64), t3.reshape(1, 64),
        fc1_w, fc1_b.reshape(1, 128), fc2_w, fc2_b.reshape(1, 10))
